# Initial kernel scaffold; baseline (speedup 1.0000x reference)
import jax
import jax.numpy as jnp
from jax.experimental import pallas as pl


def kernel(x, edge_index, batch, W1_rel, b1_rel, W1_root, bn1_gamma, bn1_beta, W2_rel, b2_rel, W2_root, W3_rel, b3_rel, W3_root, bn2_gamma, bn2_beta, lin_W, lin_b):
    raise NotImplementedError("write your pallas kernel here")


# speedup vs baseline: 7.0536x; 7.0536x over previous
import functools

import jax
import jax.numpy as jnp
from jax import lax
from jax.experimental import pallas as pl
from jax.experimental.pallas import tpu as pltpu
from jax.experimental.pallas import tpu_sc as plsc

N = 10000
E = 320000
G = 64
NC, NS, L = 2, 16, 16
NW = NC * NS

EP = 327680
IDX_ROWS = EP // 128
ROWS_PER_W = IDX_ROWS // NW
CH = 8
NCHUNK = ROWS_PER_W // CH
NP = 10112


@functools.lru_cache(maxsize=None)
def _make_scatter(D):
    NPT = NP // NS
    TAIL = N - (NS - 1) * NPT
    mesh = plsc.VectorSubcoreMesh(
        core_axis_name="c", subcore_axis_name="s", num_cores=NC, num_subcores=NS)

    @functools.partial(
        pl.kernel,
        out_type=jax.ShapeDtypeStruct((NC, N, D), jnp.float32),
        mesh=mesh,
        scratch_types=[
            pltpu.VMEM_SHARED((NP, D), jnp.float32),
            pltpu.VMEM((CH, 128), jnp.int32),
            pltpu.VMEM((CH, 128), jnp.int32),
            pltpu.VMEM((CH, 128, D), jnp.float32),
            pltpu.SemaphoreType.DMA,
        ],
        compiler_params=pltpu.CompilerParams(use_tc_tiling_on_sc=False),
    )
    def scat(y_hbm, srcp_hbm, dstp_hbm, zeros_hbm, out_hbm,
             table, src_v, dst_v, rows_v, sem):
        cid = lax.axis_index("c")
        sid = lax.axis_index("s")
        wid = cid * NS + sid
        pltpu.sync_copy(zeros_hbm.at[pl.ds(sid * NPT, NPT)],
                        table.at[pl.ds(sid * NPT, NPT)])
        plsc.subcore_barrier()
        row0 = wid * ROWS_PER_W

        def chunk(k, carry):
            r = row0 + k * CH
            pltpu.sync_copy(srcp_hbm.at[pl.ds(r, CH)], src_v)
            pltpu.sync_copy(dstp_hbm.at[pl.ds(r, CH)], dst_v)
            descs = [pltpu.async_copy(y_hbm.at[src_v.at[j]], rows_v.at[j], sem)
                     for j in range(CH)]
            for d in descs:
                d.wait()
            for j in range(CH):
                pltpu.sync_copy(rows_v.at[j], table.at[dst_v.at[j]], add=True)
            return carry

        lax.fori_loop(0, NCHUNK, chunk, 0)
        plsc.subcore_barrier()

        @pl.when(sid < NS - 1)
        def _():
            pltpu.sync_copy(table.at[pl.ds(sid * NPT, NPT)],
                            out_hbm.at[cid, pl.ds(sid * NPT, NPT)])

        @pl.when(sid == NS - 1)
        def _():
            pltpu.sync_copy(table.at[pl.ds((NS - 1) * NPT, TAIL)],
                            out_hbm.at[cid, pl.ds((NS - 1) * NPT, TAIL)])

    return scat


def _dense1(x_ref, wrelT, wrootT, b, y_ref, r_ref):
    x = x_ref[...]
    y_ref[...] = jnp.dot(x, wrelT[...], preferred_element_type=jnp.float32)
    r_ref[...] = jnp.dot(x, wrootT[...], preferred_element_type=jnp.float32) + b[...]


def _dense2(agg_ref, r_ref, gamma, beta, w2relT, w2rootT, b2, y2_ref, r2_ref):
    h = agg_ref[0] + agg_ref[1] + r_ref[...]
    mean = jnp.mean(h, axis=0, keepdims=True)
    var = jnp.mean((h - mean) ** 2, axis=0, keepdims=True)
    h = (h - mean) * lax.rsqrt(var + 1e-5) * gamma[...] + beta[...]
    h = jnp.maximum(h, 0.0)
    y2_ref[...] = jnp.dot(h, w2relT[...], preferred_element_type=jnp.float32)
    r2_ref[...] = jnp.dot(h, w2rootT[...], preferred_element_type=jnp.float32) + b2[...]


def _dense3(agg_ref, r_ref, w3relT, w3rootT, b3, y3_ref, r3_ref):
    h = jnp.maximum(agg_ref[0] + agg_ref[1] + r_ref[...], 0.0)
    y3_ref[...] = jnp.dot(h, w3relT[...], preferred_element_type=jnp.float32)
    r3_ref[...] = jnp.dot(h, w3rootT[...], preferred_element_type=jnp.float32) + b3[...]


def _dense4(agg_ref, r_ref, gamma, beta, batch_ref, linWT, linb, out_ref):
    h = agg_ref[0] + agg_ref[1] + r_ref[...]
    mean = jnp.mean(h, axis=0, keepdims=True)
    var = jnp.mean((h - mean) ** 2, axis=0, keepdims=True)
    h = (h - mean) * lax.rsqrt(var + 1e-5) * gamma[...] + beta[...]
    gids = lax.broadcasted_iota(jnp.int32, (G, N), 0)
    mask = (gids == batch_ref[...]).astype(jnp.float32)
    sums = jnp.dot(mask, h, preferred_element_type=jnp.float32)
    counts = jnp.sum(mask, axis=1, keepdims=True)
    means = sums / jnp.maximum(counts, 1.0)
    out_ref[...] = jnp.dot(means, linWT[...], preferred_element_type=jnp.float32) + linb[...]


def _tc(body, out_shape, *args):
    return pl.pallas_call(body, out_shape=out_shape)(*args)


def kernel(x, edge_index, batch, W1_rel, b1_rel, W1_root, bn1_gamma, bn1_beta,
           W2_rel, b2_rel, W2_root, W3_rel, b3_rel, W3_root,
           bn2_gamma, bn2_beta, lin_W, lin_b):
    f32 = jnp.float32
    src = edge_index[0].astype(jnp.int32)
    dst = edge_index[1].astype(jnp.int32)
    pad = EP - E
    srcp = jnp.concatenate([src, jnp.zeros((pad,), jnp.int32)]).reshape(IDX_ROWS, 128)
    dstp = jnp.concatenate([dst, jnp.full((pad,), N, jnp.int32)]).reshape(IDX_ROWS, 128)
    batch32 = batch.astype(jnp.int32).reshape(1, N)
    zeros64 = jnp.zeros((NP, 64), f32)
    zeros32 = jnp.zeros((NP, 32), f32)

    w3relT = jnp.zeros((32, 32), f32).at[:, :20].set(W3_rel.T)
    w3rootT = jnp.zeros((32, 32), f32).at[:, :20].set(W3_root.T)
    b3p = jnp.zeros((1, 32), f32).at[0, :20].set(b3_rel)
    g2p = jnp.zeros((1, 32), f32).at[0, :20].set(bn2_gamma)
    be2p = jnp.zeros((1, 32), f32).at[0, :20].set(bn2_beta)
    linWT = jnp.zeros((32, 11), f32).at[:20, :].set(lin_W.T)

    sd = jax.ShapeDtypeStruct
    y1, r1 = _tc(_dense1, (sd((N, 64), f32), sd((N, 64), f32)),
                 x, W1_rel.T, W1_root.T, b1_rel.reshape(1, 64))
    agg1 = _make_scatter(64)(y1, srcp, dstp, zeros64)
    y2, r2 = _tc(_dense2, (sd((N, 32), f32), sd((N, 32), f32)),
                 agg1, r1, bn1_gamma.reshape(1, 64), bn1_beta.reshape(1, 64),
                 W2_rel.T, W2_root.T, b2_rel.reshape(1, 32))
    agg2 = _make_scatter(32)(y2, srcp, dstp, zeros32)
    y3, r3 = _tc(_dense3, (sd((N, 32), f32), sd((N, 32), f32)),
                 agg2, r2, w3relT, w3rootT, b3p)
    agg3 = _make_scatter(32)(y3, srcp, dstp, zeros32)
    out = _tc(_dense4, sd((G, 11), f32),
              agg3, r3, g2p, be2p, batch32, linWT, lin_b.reshape(1, 11))
    return out


# speedup vs baseline: 7.4535x; 1.0567x over previous
import functools

import jax
import jax.numpy as jnp
from jax import lax
from jax.experimental import pallas as pl
from jax.experimental.pallas import tpu as pltpu
from jax.experimental.pallas import tpu_sc as plsc

N = 10000
E = 320000
G = 64
NC, NS, L = 2, 16, 16
NW = NC * NS

EP = 327680
IDX_ROWS = EP // 128
ROWS_PER_W = IDX_ROWS // NW
CH = 4
NGROUP = ROWS_PER_W // CH
NP = 10112


@functools.lru_cache(maxsize=None)
def _make_scatter(D):
    NPT = NP // NS
    TAIL = N - (NS - 1) * NPT
    mesh = plsc.VectorSubcoreMesh(
        core_axis_name="c", subcore_axis_name="s", num_cores=NC, num_subcores=NS)

    @functools.partial(
        pl.kernel,
        out_type=jax.ShapeDtypeStruct((NC, N, D), jnp.float32),
        mesh=mesh,
        scratch_types=[
            pltpu.VMEM_SHARED((NP, D), jnp.float32),
            pltpu.VMEM((ROWS_PER_W, 128), jnp.int32),
            pltpu.VMEM((ROWS_PER_W, 128), jnp.int32),
            pltpu.VMEM((2, CH, 128, D), jnp.float32),
            pltpu.SemaphoreType.DMA,
            pltpu.SemaphoreType.DMA,
            pltpu.SemaphoreType.DMA,
        ],
        compiler_params=pltpu.CompilerParams(use_tc_tiling_on_sc=False),
    )
    def scat(y_hbm, srcp_hbm, dstp_hbm, zeros_hbm, out_hbm,
             table, src_v, dst_v, rows_v, zsem, gsem, ssem):
        cid = lax.axis_index("c")
        sid = lax.axis_index("s")
        wid = cid * NS + sid
        zd = pltpu.async_copy(zeros_hbm.at[pl.ds(sid * NPT, NPT)],
                              table.at[pl.ds(sid * NPT, NPT)], zsem)
        row0 = wid * ROWS_PER_W
        pltpu.sync_copy(srcp_hbm.at[pl.ds(row0, ROWS_PER_W)], src_v)
        pltpu.sync_copy(dstp_hbm.at[pl.ds(row0, ROWS_PER_W)], dst_v)
        gd = {}
        for b in range(CH):
            gd[(0, b)] = pltpu.async_copy(y_hbm.at[src_v.at[b]],
                                          rows_v.at[0, b], gsem)
        zd.wait()
        plsc.subcore_barrier()

        sd = {}
        for g in range(NGROUP):
            buf = g % 2
            for b in range(CH):
                gd.pop((g, b)).wait()
            for b in range(CH):
                sd[(g, b)] = pltpu.async_copy(
                    rows_v.at[buf, b], table.at[dst_v.at[g * CH + b]],
                    ssem, add=True)
            if g + 1 < NGROUP:
                for b in range(CH):
                    gd[(g + 1, b)] = pltpu.async_copy(
                        y_hbm.at[src_v.at[(g + 1) * CH + b]],
                        rows_v.at[1 - buf, b], gsem)
            for b in range(CH):
                sd.pop((g, b)).wait()
        plsc.subcore_barrier()

        @pl.when(sid < NS - 1)
        def _():
            pltpu.sync_copy(table.at[pl.ds(sid * NPT, NPT)],
                            out_hbm.at[cid, pl.ds(sid * NPT, NPT)])

        @pl.when(sid == NS - 1)
        def _():
            pltpu.sync_copy(table.at[pl.ds((NS - 1) * NPT, TAIL)],
                            out_hbm.at[cid, pl.ds((NS - 1) * NPT, TAIL)])

    return scat


def _dense1(x_ref, wrelT, wrootT, b, y_ref, r_ref):
    x = x_ref[...]
    y_ref[...] = jnp.dot(x, wrelT[...], preferred_element_type=jnp.float32, precision=lax.Precision.HIGHEST)
    r_ref[...] = jnp.dot(x, wrootT[...], preferred_element_type=jnp.float32, precision=lax.Precision.HIGHEST) + b[...]


def _dense2(agg_ref, r_ref, gamma, beta, w2relT, w2rootT, b2, y2_ref, r2_ref):
    h = agg_ref[0] + agg_ref[1] + r_ref[...]
    mean = jnp.mean(h, axis=0, keepdims=True)
    var = jnp.mean((h - mean) ** 2, axis=0, keepdims=True)
    h = (h - mean) * lax.rsqrt(var + 1e-5) * gamma[...] + beta[...]
    h = jnp.maximum(h, 0.0)
    y2_ref[...] = jnp.dot(h, w2relT[...], preferred_element_type=jnp.float32, precision=lax.Precision.HIGHEST)
    r2_ref[...] = jnp.dot(h, w2rootT[...], preferred_element_type=jnp.float32, precision=lax.Precision.HIGHEST) + b2[...]


def _dense3(agg_ref, r_ref, w3relT, w3rootT, b3, y3_ref, r3_ref):
    h = jnp.maximum(agg_ref[0] + agg_ref[1] + r_ref[...], 0.0)
    y3_ref[...] = jnp.dot(h, w3relT[...], preferred_element_type=jnp.float32, precision=lax.Precision.HIGHEST)
    r3_ref[...] = jnp.dot(h, w3rootT[...], preferred_element_type=jnp.float32, precision=lax.Precision.HIGHEST) + b3[...]


def _dense4(agg_ref, r_ref, gamma, beta, batch_ref, linWT, linb, out_ref):
    h = agg_ref[0] + agg_ref[1] + r_ref[...]
    mean = jnp.mean(h, axis=0, keepdims=True)
    var = jnp.mean((h - mean) ** 2, axis=0, keepdims=True)
    h = (h - mean) * lax.rsqrt(var + 1e-5) * gamma[...] + beta[...]
    gids = lax.broadcasted_iota(jnp.int32, (G, N), 0)
    mask = (gids == batch_ref[...]).astype(jnp.float32)
    sums = jnp.dot(mask, h, preferred_element_type=jnp.float32, precision=lax.Precision.HIGHEST)
    counts = jnp.sum(mask, axis=1, keepdims=True)
    means = sums / jnp.maximum(counts, 1.0)
    out_ref[...] = jnp.dot(means, linWT[...], preferred_element_type=jnp.float32, precision=lax.Precision.HIGHEST) + linb[...]


def _tc(body, out_shape, *args):
    return pl.pallas_call(body, out_shape=out_shape)(*args)


def kernel(x, edge_index, batch, W1_rel, b1_rel, W1_root, bn1_gamma, bn1_beta,
           W2_rel, b2_rel, W2_root, W3_rel, b3_rel, W3_root,
           bn2_gamma, bn2_beta, lin_W, lin_b):
    f32 = jnp.float32
    src = edge_index[0].astype(jnp.int32)
    dst = edge_index[1].astype(jnp.int32)
    pad = EP - E
    srcp = jnp.concatenate([src, jnp.zeros((pad,), jnp.int32)]).reshape(IDX_ROWS, 128)
    dstp = jnp.concatenate([dst, jnp.full((pad,), N, jnp.int32)]).reshape(IDX_ROWS, 128)
    batch32 = batch.astype(jnp.int32).reshape(1, N)
    zeros64 = jnp.zeros((NP, 64), f32)
    zeros32 = jnp.zeros((NP, 32), f32)

    w3relT = jnp.zeros((32, 32), f32).at[:, :20].set(W3_rel.T)
    w3rootT = jnp.zeros((32, 32), f32).at[:, :20].set(W3_root.T)
    b3p = jnp.zeros((1, 32), f32).at[0, :20].set(b3_rel)
    g2p = jnp.zeros((1, 32), f32).at[0, :20].set(bn2_gamma)
    be2p = jnp.zeros((1, 32), f32).at[0, :20].set(bn2_beta)
    linWT = jnp.zeros((32, 11), f32).at[:20, :].set(lin_W.T)

    sd = jax.ShapeDtypeStruct
    y1, r1 = _tc(_dense1, (sd((N, 64), f32), sd((N, 64), f32)),
                 x, W1_rel.T, W1_root.T, b1_rel.reshape(1, 64))
    agg1 = _make_scatter(64)(y1, srcp, dstp, zeros64)
    y2, r2 = _tc(_dense2, (sd((N, 32), f32), sd((N, 32), f32)),
                 agg1, r1, bn1_gamma.reshape(1, 64), bn1_beta.reshape(1, 64),
                 W2_rel.T, W2_root.T, b2_rel.reshape(1, 32))
    agg2 = _make_scatter(32)(y2, srcp, dstp, zeros32)
    y3, r3 = _tc(_dense3, (sd((N, 32), f32), sd((N, 32), f32)),
                 agg2, r2, w3relT, w3rootT, b3p)
    agg3 = _make_scatter(32)(y3, srcp, dstp, zeros32)
    out = _tc(_dense4, sd((G, 11), f32),
              agg3, r3, g2p, be2p, batch32, linWT, lin_b.reshape(1, 11))
    return out


# speedup vs baseline: 7.7713x; 1.0426x over previous
import functools

import jax
import jax.numpy as jnp
from jax import lax
from jax.experimental import pallas as pl
from jax.experimental.pallas import tpu as pltpu
from jax.experimental.pallas import tpu_sc as plsc

N = 10000
E = 320000
G = 64
NC, NS, L = 2, 16, 16
NW = NC * NS

EP = 327680
IDX_ROWS = EP // 128
CH = 4
NP = 10112
R_CORE = (120, 40)
SEG = 40


@functools.lru_cache(maxsize=None)
def _make_scatter(D):
    NPT = NP // NS
    TAIL = N - (NS - 1) * NPT
    mesh = plsc.VectorSubcoreMesh(
        core_axis_name="c", subcore_axis_name="s", num_cores=NC, num_subcores=NS)

    @functools.partial(
        pl.kernel,
        out_type=jax.ShapeDtypeStruct((NC, N, D), jnp.float32),
        mesh=mesh,
        scratch_types=[
            pltpu.VMEM_SHARED((NP, D), jnp.float32),
            pltpu.VMEM((SEG, 128), jnp.int32),
            pltpu.VMEM((SEG, 128), jnp.int32),
            pltpu.VMEM((2, CH, 128, D), jnp.float32),
            pltpu.SemaphoreType.DMA,
            pltpu.SemaphoreType.DMA,
            pltpu.SemaphoreType.DMA,
        ],
        compiler_params=pltpu.CompilerParams(use_tc_tiling_on_sc=False),
    )
    def scat(y_hbm, srcp_hbm, dstp_hbm, zeros_hbm, out_hbm,
             table, src_v, dst_v, rows_v, zsem, gsem, ssem):
        cid = lax.axis_index("c")
        sid = lax.axis_index("s")
        zd = pltpu.async_copy(zeros_hbm.at[pl.ds(sid * NPT, NPT)],
                              table.at[pl.ds(sid * NPT, NPT)], zsem)
        zd.wait()
        plsc.subcore_barrier()

        def segment(base):
            ngroup = SEG // CH
            pltpu.sync_copy(srcp_hbm.at[pl.ds(base, SEG)], src_v)
            pltpu.sync_copy(dstp_hbm.at[pl.ds(base, SEG)], dst_v)
            gd = {}
            for b in range(CH):
                gd[(0, b)] = pltpu.async_copy(y_hbm.at[src_v.at[b]],
                                              rows_v.at[0, b], gsem)
            sd = {}
            for g in range(ngroup):
                buf = g % 2
                for b in range(CH):
                    gd.pop((g, b)).wait()
                for b in range(CH):
                    sd[(g, b)] = pltpu.async_copy(
                        rows_v.at[buf, b], table.at[dst_v.at[g * CH + b]],
                        ssem, add=True)
                if g + 1 < ngroup:
                    for b in range(CH):
                        gd[(g + 1, b)] = pltpu.async_copy(
                            y_hbm.at[src_v.at[(g + 1) * CH + b]],
                            rows_v.at[1 - buf, b], gsem)
                for b in range(CH):
                    sd.pop((g, b)).wait()

        @pl.when(cid == 0)
        def _():
            for s in range(R_CORE[0] // SEG):
                segment(sid * R_CORE[0] + s * SEG)

        @pl.when(cid == 1)
        def _():
            for s in range(R_CORE[1] // SEG):
                segment(NS * R_CORE[0] + sid * R_CORE[1] + s * SEG)

        plsc.subcore_barrier()

        @pl.when(sid < NS - 1)
        def _():
            pltpu.sync_copy(table.at[pl.ds(sid * NPT, NPT)],
                            out_hbm.at[cid, pl.ds(sid * NPT, NPT)])

        @pl.when(sid == NS - 1)
        def _():
            pltpu.sync_copy(table.at[pl.ds((NS - 1) * NPT, TAIL)],
                            out_hbm.at[cid, pl.ds((NS - 1) * NPT, TAIL)])

    return scat


def _dense1(x_ref, wrelT, wrootT, b, y_ref, r_ref):
    x = x_ref[...]
    y_ref[...] = jnp.dot(x, wrelT[...], preferred_element_type=jnp.float32, precision=lax.Precision.HIGHEST)
    r_ref[...] = jnp.dot(x, wrootT[...], preferred_element_type=jnp.float32, precision=lax.Precision.HIGHEST) + b[...]


def _dense2(agg_ref, r_ref, gamma, beta, w2relT, w2rootT, b2, y2_ref, r2_ref):
    h = agg_ref[0] + agg_ref[1] + r_ref[...]
    mean = jnp.mean(h, axis=0, keepdims=True)
    var = jnp.mean((h - mean) ** 2, axis=0, keepdims=True)
    h = (h - mean) * lax.rsqrt(var + 1e-5) * gamma[...] + beta[...]
    h = jnp.maximum(h, 0.0)
    y2_ref[...] = jnp.dot(h, w2relT[...], preferred_element_type=jnp.float32, precision=lax.Precision.HIGHEST)
    r2_ref[...] = jnp.dot(h, w2rootT[...], preferred_element_type=jnp.float32, precision=lax.Precision.HIGHEST) + b2[...]


def _dense3(agg_ref, r_ref, w3relT, w3rootT, b3, y3_ref, r3_ref):
    h = jnp.maximum(agg_ref[0] + agg_ref[1] + r_ref[...], 0.0)
    y3_ref[...] = jnp.dot(h, w3relT[...], preferred_element_type=jnp.float32, precision=lax.Precision.HIGHEST)
    r3_ref[...] = jnp.dot(h, w3rootT[...], preferred_element_type=jnp.float32, precision=lax.Precision.HIGHEST) + b3[...]


def _dense4(agg_ref, r_ref, gamma, beta, batch_ref, linWT, linb, out_ref):
    h = agg_ref[0] + agg_ref[1] + r_ref[...]
    mean = jnp.mean(h, axis=0, keepdims=True)
    var = jnp.mean((h - mean) ** 2, axis=0, keepdims=True)
    h = (h - mean) * lax.rsqrt(var + 1e-5) * gamma[...] + beta[...]
    gids = lax.broadcasted_iota(jnp.int32, (G, N), 0)
    mask = (gids == batch_ref[...]).astype(jnp.float32)
    sums = jnp.dot(mask, h, preferred_element_type=jnp.float32, precision=lax.Precision.HIGHEST)
    counts = jnp.sum(mask, axis=1, keepdims=True)
    means = sums / jnp.maximum(counts, 1.0)
    out_ref[...] = jnp.dot(means, linWT[...], preferred_element_type=jnp.float32, precision=lax.Precision.HIGHEST) + linb[...]


def _tc(body, out_shape, *args):
    return pl.pallas_call(body, out_shape=out_shape)(*args)


def kernel(x, edge_index, batch, W1_rel, b1_rel, W1_root, bn1_gamma, bn1_beta,
           W2_rel, b2_rel, W2_root, W3_rel, b3_rel, W3_root,
           bn2_gamma, bn2_beta, lin_W, lin_b):
    f32 = jnp.float32
    src = edge_index[0].astype(jnp.int32)
    dst = edge_index[1].astype(jnp.int32)
    pad = EP - E
    srcp = jnp.concatenate([src, jnp.zeros((pad,), jnp.int32)]).reshape(IDX_ROWS, 128)
    dstp = jnp.concatenate([dst, jnp.full((pad,), N, jnp.int32)]).reshape(IDX_ROWS, 128)
    batch32 = batch.astype(jnp.int32).reshape(1, N)
    zeros64 = jnp.zeros((NP, 64), f32)
    zeros32 = jnp.zeros((NP, 32), f32)

    w3relT = jnp.zeros((32, 32), f32).at[:, :20].set(W3_rel.T)
    w3rootT = jnp.zeros((32, 32), f32).at[:, :20].set(W3_root.T)
    b3p = jnp.zeros((1, 32), f32).at[0, :20].set(b3_rel)
    g2p = jnp.zeros((1, 32), f32).at[0, :20].set(bn2_gamma)
    be2p = jnp.zeros((1, 32), f32).at[0, :20].set(bn2_beta)
    linWT = jnp.zeros((32, 11), f32).at[:20, :].set(lin_W.T)

    sd = jax.ShapeDtypeStruct
    y1, r1 = _tc(_dense1, (sd((N, 64), f32), sd((N, 64), f32)),
                 x, W1_rel.T, W1_root.T, b1_rel.reshape(1, 64))
    agg1 = _make_scatter(64)(y1, srcp, dstp, zeros64)
    y2, r2 = _tc(_dense2, (sd((N, 32), f32), sd((N, 32), f32)),
                 agg1, r1, bn1_gamma.reshape(1, 64), bn1_beta.reshape(1, 64),
                 W2_rel.T, W2_root.T, b2_rel.reshape(1, 32))
    agg2 = _make_scatter(32)(y2, srcp, dstp, zeros32)
    y3, r3 = _tc(_dense3, (sd((N, 32), f32), sd((N, 32), f32)),
                 agg2, r2, w3relT, w3rootT, b3p)
    agg3 = _make_scatter(32)(y3, srcp, dstp, zeros32)
    out = _tc(_dense4, sd((G, 11), f32),
              agg3, r3, g2p, be2p, batch32, linWT, lin_b.reshape(1, 11))
    return out


# speedup vs baseline: 14.0069x; 1.8024x over previous
import functools

import jax
import jax.numpy as jnp
from jax import lax
from jax.experimental import pallas as pl
from jax.experimental.pallas import tpu as pltpu
from jax.experimental.pallas import tpu_sc as plsc

N = 10000
E = 320000
G = 64
NC, NS, L = 2, 16, 16
NW = NC * NS

EP = 327680
IDX_ROWS = EP // 128
CH = 4
NP = 10112
R_CORE = (80, 80)
SEG = 40


@functools.lru_cache(maxsize=None)
def _make_scatter(D):
    NPT = NP // NS
    TAIL = N - (NS - 1) * NPT
    mesh = plsc.VectorSubcoreMesh(
        core_axis_name="c", subcore_axis_name="s", num_cores=NC, num_subcores=NS)

    CHD = 2 if D > 32 else CH

    @functools.partial(
        pl.kernel,
        out_type=jax.ShapeDtypeStruct((NC, N, D), jnp.float32),
        mesh=mesh,
        scratch_types=[
            pltpu.VMEM_SHARED((NP, D), jnp.float32),
            pltpu.VMEM_SHARED((NP, D), jnp.float32),
            pltpu.VMEM((SEG, 128), jnp.int32),
            pltpu.VMEM((SEG, 128), jnp.int32),
            pltpu.VMEM((2, CHD, 128, D), jnp.float32),
            pltpu.SemaphoreType.DMA,
            pltpu.SemaphoreType.DMA,
            pltpu.SemaphoreType.DMA,
        ],
        compiler_params=pltpu.CompilerParams(use_tc_tiling_on_sc=False),
    )
    def scat(y_hbm, srcp_hbm, dstp_hbm, zeros_hbm, out_hbm,
             table, ytab, src_v, dst_v, rows_v, zsem, gsem, ssem):
        cid = lax.axis_index("c")
        sid = lax.axis_index("s")
        zd = pltpu.async_copy(zeros_hbm.at[pl.ds(sid * NPT, NPT)],
                              table.at[pl.ds(sid * NPT, NPT)], zsem)

        @pl.when(sid < NS - 1)
        def _():
            pltpu.sync_copy(y_hbm.at[pl.ds(sid * NPT, NPT)],
                            ytab.at[pl.ds(sid * NPT, NPT)])

        @pl.when(sid == NS - 1)
        def _():
            pltpu.sync_copy(y_hbm.at[pl.ds((NS - 1) * NPT, TAIL)],
                            ytab.at[pl.ds((NS - 1) * NPT, TAIL)])

        zd.wait()
        plsc.subcore_barrier()

        def segment(base):
            ngroup = SEG // CHD
            pltpu.sync_copy(srcp_hbm.at[pl.ds(base, SEG)], src_v)
            pltpu.sync_copy(dstp_hbm.at[pl.ds(base, SEG)], dst_v)
            gd = {}
            for b in range(CHD):
                gd[(0, b)] = pltpu.async_copy(ytab.at[src_v.at[b]],
                                              rows_v.at[0, b], gsem)
            sd = {}
            for g in range(ngroup):
                buf = g % 2
                for b in range(CHD):
                    gd.pop((g, b)).wait()
                for b in range(CHD):
                    sd[(g, b)] = pltpu.async_copy(
                        rows_v.at[buf, b], table.at[dst_v.at[g * CHD + b]],
                        ssem, add=True)
                if g + 1 < ngroup:
                    for b in range(CHD):
                        gd[(g + 1, b)] = pltpu.async_copy(
                            ytab.at[src_v.at[(g + 1) * CHD + b]],
                            rows_v.at[1 - buf, b], gsem)
                for b in range(CHD):
                    sd.pop((g, b)).wait()

        @pl.when(cid == 0)
        def _():
            for s in range(R_CORE[0] // SEG):
                segment(sid * R_CORE[0] + s * SEG)

        @pl.when(cid == 1)
        def _():
            for s in range(R_CORE[1] // SEG):
                segment(NS * R_CORE[0] + sid * R_CORE[1] + s * SEG)

        plsc.subcore_barrier()

        @pl.when(sid < NS - 1)
        def _():
            pltpu.sync_copy(table.at[pl.ds(sid * NPT, NPT)],
                            out_hbm.at[cid, pl.ds(sid * NPT, NPT)])

        @pl.when(sid == NS - 1)
        def _():
            pltpu.sync_copy(table.at[pl.ds((NS - 1) * NPT, TAIL)],
                            out_hbm.at[cid, pl.ds((NS - 1) * NPT, TAIL)])

    return scat


def _dense1(x_ref, wrelT, wrootT, b, y_ref, r_ref):
    x = x_ref[...]
    y_ref[...] = jnp.dot(x, wrelT[...], preferred_element_type=jnp.float32, precision=lax.Precision.HIGHEST)
    r_ref[...] = jnp.dot(x, wrootT[...], preferred_element_type=jnp.float32, precision=lax.Precision.HIGHEST) + b[...]


def _dense2(agg_ref, r_ref, gamma, beta, w2relT, w2rootT, b2, y2_ref, r2_ref):
    h = agg_ref[0] + agg_ref[1] + r_ref[...]
    mean = jnp.mean(h, axis=0, keepdims=True)
    var = jnp.mean((h - mean) ** 2, axis=0, keepdims=True)
    h = (h - mean) * lax.rsqrt(var + 1e-5) * gamma[...] + beta[...]
    h = jnp.maximum(h, 0.0)
    y2_ref[...] = jnp.dot(h, w2relT[...], preferred_element_type=jnp.float32, precision=lax.Precision.HIGHEST)
    r2_ref[...] = jnp.dot(h, w2rootT[...], preferred_element_type=jnp.float32, precision=lax.Precision.HIGHEST) + b2[...]


def _dense3(agg_ref, r_ref, w3relT, w3rootT, b3, y3_ref, r3_ref):
    h = jnp.maximum(agg_ref[0] + agg_ref[1] + r_ref[...], 0.0)
    y3_ref[...] = jnp.dot(h, w3relT[...], preferred_element_type=jnp.float32, precision=lax.Precision.HIGHEST)
    r3_ref[...] = jnp.dot(h, w3rootT[...], preferred_element_type=jnp.float32, precision=lax.Precision.HIGHEST) + b3[...]


def _dense4(agg_ref, r_ref, gamma, beta, batch_ref, linWT, linb, out_ref):
    h = agg_ref[0] + agg_ref[1] + r_ref[...]
    mean = jnp.mean(h, axis=0, keepdims=True)
    var = jnp.mean((h - mean) ** 2, axis=0, keepdims=True)
    h = (h - mean) * lax.rsqrt(var + 1e-5) * gamma[...] + beta[...]
    gids = lax.broadcasted_iota(jnp.int32, (G, N), 0)
    mask = (gids == batch_ref[...]).astype(jnp.float32)
    sums = jnp.dot(mask, h, preferred_element_type=jnp.float32, precision=lax.Precision.HIGHEST)
    counts = jnp.sum(mask, axis=1, keepdims=True)
    means = sums / jnp.maximum(counts, 1.0)
    out_ref[...] = jnp.dot(means, linWT[...], preferred_element_type=jnp.float32, precision=lax.Precision.HIGHEST) + linb[...]


def _tc(body, out_shape, *args):
    return pl.pallas_call(body, out_shape=out_shape)(*args)


def kernel(x, edge_index, batch, W1_rel, b1_rel, W1_root, bn1_gamma, bn1_beta,
           W2_rel, b2_rel, W2_root, W3_rel, b3_rel, W3_root,
           bn2_gamma, bn2_beta, lin_W, lin_b):
    f32 = jnp.float32
    src = edge_index[0].astype(jnp.int32)
    dst = edge_index[1].astype(jnp.int32)
    pad = EP - E
    srcp = jnp.concatenate([src, jnp.zeros((pad,), jnp.int32)]).reshape(IDX_ROWS, 128)
    dstp = jnp.concatenate([dst, jnp.full((pad,), N, jnp.int32)]).reshape(IDX_ROWS, 128)
    batch32 = batch.astype(jnp.int32).reshape(1, N)
    zeros64 = jnp.zeros((NP, 64), f32)
    zeros32 = jnp.zeros((NP, 32), f32)

    w3relT = jnp.zeros((32, 32), f32).at[:, :20].set(W3_rel.T)
    w3rootT = jnp.zeros((32, 32), f32).at[:, :20].set(W3_root.T)
    b3p = jnp.zeros((1, 32), f32).at[0, :20].set(b3_rel)
    g2p = jnp.zeros((1, 32), f32).at[0, :20].set(bn2_gamma)
    be2p = jnp.zeros((1, 32), f32).at[0, :20].set(bn2_beta)
    linWT = jnp.zeros((32, 11), f32).at[:20, :].set(lin_W.T)

    sd = jax.ShapeDtypeStruct
    y1, r1 = _tc(_dense1, (sd((N, 64), f32), sd((N, 64), f32)),
                 x, W1_rel.T, W1_root.T, b1_rel.reshape(1, 64))
    agg1 = _make_scatter(64)(y1, srcp, dstp, zeros64)
    y2, r2 = _tc(_dense2, (sd((N, 32), f32), sd((N, 32), f32)),
                 agg1, r1, bn1_gamma.reshape(1, 64), bn1_beta.reshape(1, 64),
                 W2_rel.T, W2_root.T, b2_rel.reshape(1, 32))
    agg2 = _make_scatter(32)(y2, srcp, dstp, zeros32)
    y3, r3 = _tc(_dense3, (sd((N, 32), f32), sd((N, 32), f32)),
                 agg2, r2, w3relT, w3rootT, b3p)
    agg3 = _make_scatter(32)(y3, srcp, dstp, zeros32)
    out = _tc(_dense4, sd((G, 11), f32),
              agg3, r3, g2p, be2p, batch32, linWT, lin_b.reshape(1, 11))
    return out
